# Initial kernel scaffold; baseline (speedup 1.0000x reference)
import jax
import jax.numpy as jnp
from jax.experimental import pallas as pl


def kernel(src, dst, t, x_pad, t_pad, node_bias, emb, beta):
    raise NotImplementedError("write your pallas kernel here")


# speedup vs baseline: 8.0692x; 8.0692x over previous
import functools

import jax
import jax.numpy as jnp
from jax import lax
from jax.experimental import pallas as pl
from jax.experimental.pallas import tpu as pltpu
from jax.experimental.pallas import tpu_sc as plsc

NC = 2
NS = 16
NW = NC * NS
CHUNK = 128
GRP = 16


def _sc_kernel(src, dst, node_bias, emb):
    B = src.shape[0]
    NN, EMB = emb.shape
    total_chunks = B // CHUNK
    mesh = plsc.VectorSubcoreMesh(core_axis_name="c", subcore_axis_name="s")

    def body(src_hbm, dst_hbm, bias_hbm, emb_hbm, sbias_hbm, sdot_hbm,
             bias_v, sidx_v, didx_v, srows_v, drows_v, sb_v, sd_v, sem):
        wid = lax.axis_index("s") * NC + lax.axis_index("c")
        pltpu.sync_copy(bias_hbm, bias_v)
        nchunks = (total_chunks - wid + NW - 1) // NW

        def chunk_body(i, _):
            base = (wid + i * NW) * CHUNK
            pltpu.sync_copy(src_hbm.at[pl.ds(base, CHUNK)], sidx_v)
            pltpu.sync_copy(dst_hbm.at[pl.ds(base, CHUNK)], didx_v)
            c1 = pltpu.async_copy(emb_hbm.at[sidx_v], srows_v, sem)
            c2 = pltpu.async_copy(emb_hbm.at[didx_v], drows_v, sem)
            c1.wait()
            c2.wait()

            def grp_body(g, _):
                off = g * GRP
                sv = sidx_v[pl.ds(off, GRP)]
                dv = didx_v[pl.ds(off, GRP)]
                bs = (plsc.load_gather(bias_v, [sv])
                      + plsc.load_gather(bias_v, [dv]))
                sb_v[pl.ds(off, GRP)] = bs
                rowid = lax.iota(jnp.int32, GRP) + off
                acc = jnp.zeros((GRP,), jnp.float32)
                for d in range(EMB):
                    dsplat = jnp.full((GRP,), d, jnp.int32)
                    a = plsc.load_gather(srows_v, [rowid, dsplat])
                    b = plsc.load_gather(drows_v, [rowid, dsplat])
                    acc = acc + a * b
                sd_v[pl.ds(off, GRP)] = acc
                return 0

            lax.fori_loop(0, CHUNK // GRP, grp_body, 0)
            pltpu.sync_copy(sb_v, sbias_hbm.at[pl.ds(base, CHUNK)])
            pltpu.sync_copy(sd_v, sdot_hbm.at[pl.ds(base, CHUNK)])
            return 0

        lax.fori_loop(0, nchunks, chunk_body, 0)

    f = pl.kernel(
        body,
        out_type=(jax.ShapeDtypeStruct((B,), jnp.float32),
                  jax.ShapeDtypeStruct((B,), jnp.float32)),
        mesh=mesh,
        compiler_params=pltpu.CompilerParams(
            needs_layout_passes=False,
            use_tc_tiling_on_sc=False,
        ),
        scratch_types=[
            pltpu.VMEM((NN,), jnp.float32),
            pltpu.VMEM((CHUNK,), jnp.int32),
            pltpu.VMEM((CHUNK,), jnp.int32),
            pltpu.VMEM((CHUNK, EMB), jnp.float32),
            pltpu.VMEM((CHUNK, EMB), jnp.float32),
            pltpu.VMEM((CHUNK,), jnp.float32),
            pltpu.VMEM((CHUNK,), jnp.float32),
            pltpu.SemaphoreType.DMA,
        ],
    )
    return f(src, dst, node_bias, emb)


_RR = 256
_CC = 625


def _scan_body(beta_sm, t_ref, tp_ref, xp_ref, fac_ref):
    tv = t_ref[...]
    tp = tp_ref[...]
    xp = xp_ref[...]
    mask = tp < tv[:, :, None]
    idxs = lax.broadcasted_iota(jnp.int32, tp.shape, 2)
    idx = jnp.max(jnp.where(mask, idxs, -1), axis=2)
    found = idx >= 0
    onehot = idxs == idx[:, :, None]
    x_last = jnp.sum(jnp.where(onehot, xp, 0.0), axis=2)
    t_last = jnp.sum(jnp.where(onehot, tp, 0.0), axis=2)
    sb = jax.nn.softplus(beta_sm[0, 0])
    fac = jax.nn.sigmoid(x_last) * jnp.exp(-sb * (tv - t_last))
    fac_ref[...] = jnp.where(found, fac, 0.0)


def _scan_tc(t, x_pad, t_pad, beta):
    B, L = t_pad.shape
    br = 8
    grid = _RR // br
    beta2 = jnp.reshape(beta, (1, 1))
    t2 = jnp.reshape(t, (_RR, _CC))
    tp3 = jnp.reshape(t_pad, (_RR, _CC, L))
    xp3 = jnp.reshape(x_pad, (_RR, _CC, L))
    fac = pl.pallas_call(
        _scan_body,
        grid=(grid,),
        in_specs=[
            pl.BlockSpec(memory_space=pltpu.SMEM),
            pl.BlockSpec((br, _CC), lambda i: (i, 0)),
            pl.BlockSpec((br, _CC, L), lambda i: (i, 0, 0)),
            pl.BlockSpec((br, _CC, L), lambda i: (i, 0, 0)),
        ],
        out_specs=pl.BlockSpec((br, _CC), lambda i: (i, 0)),
        out_shape=jax.ShapeDtypeStruct((_RR, _CC), jnp.float32),
    )(beta2, t2, tp3, xp3)
    return jnp.reshape(fac, (B,))


def _combine_body(sbias_ref, sdot_ref, fac_ref, out_ref):
    out_ref[...] = (jax.nn.softplus(sbias_ref[...])
                    + jax.nn.softplus(sdot_ref[...]) * fac_ref[...])


def _combine_tc(sbias, sdot, fac):
    B = sbias.shape[0]
    br = 64
    grid = _RR // br
    spec = pl.BlockSpec((br, _CC), lambda i: (i, 0))
    out = pl.pallas_call(
        _combine_body,
        grid=(grid,),
        in_specs=[spec, spec, spec],
        out_specs=spec,
        out_shape=jax.ShapeDtypeStruct((_RR, _CC), jnp.float32),
    )(jnp.reshape(sbias, (_RR, _CC)), jnp.reshape(sdot, (_RR, _CC)),
      jnp.reshape(fac, (_RR, _CC)))
    return jnp.reshape(out, (B,))


def kernel(src, dst, t, x_pad, t_pad, node_bias, emb, beta):
    src = src.astype(jnp.int32)
    dst = dst.astype(jnp.int32)
    sbias, sdot = _sc_kernel(src, dst, node_bias, emb)
    fac = _scan_tc(t, x_pad, t_pad, beta)
    return _combine_tc(sbias, sdot, fac)
